# baseline (speedup 1.0000x reference)
import functools

import jax
import jax.numpy as jnp
from jax import lax
from jax.experimental import pallas as pl
from jax.experimental.pallas import tpu as pltpu
from jax.experimental.pallas import tpu_sc as plsc

NC = 2
NS = 16
NL = 16
NW = NC * NS

CHUNK = 128


NBUF = 2

CORE_CHUNKS = (56, 76)


def _spmm_sc_kernel(cout, xt_hbm, cols_hbm, rows_hbm, vals_hbm,
                    out_hbm, cols_v, rows_v, vals_v, g_bufs, accum,
                    sem_g, sem_s):
    c = lax.axis_index("c")
    s = lax.axis_index("s")
    wid = c * NS + s
    n_chunks = jnp.where(c == 0, CORE_CHUNKS[0], CORE_CHUNKS[1])
    g_a = g_bufs[0]
    b = g_a.shape[1]

    zero16 = jnp.zeros((NL,), jnp.float32)

    def _zrow(i, _):
        for q in range(b // NL):
            g_a[i, pl.ds(q * NL, NL)] = zero16
        return 0
    lax.fori_loop(0, CHUNK, _zrow, 0)

    rows_per_tile = cout // NS
    for k in range(rows_per_tile // CHUNK):
        pltpu.sync_copy(g_a, accum.at[pl.ds(s * rows_per_tile + k * CHUNK, CHUNK)])
    plsc.subcore_barrier()

    pltpu.sync_copy(cols_hbm.at[wid], cols_v)
    pltpu.sync_copy(rows_hbm.at[wid], rows_v)
    pltpu.sync_copy(vals_hbm.at[wid], vals_v)

    def _gather(ch, g, sem):
        pltpu.async_copy(xt_hbm.at[cols_v.at[ch]], g, sem)

    def _gather_wait(ch, g, sem):
        pltpu.make_async_copy(xt_hbm.at[cols_v.at[ch]], g, sem).wait()

    def _scatter(ch, g, sem):
        pltpu.async_copy(g, accum.at[rows_v.at[ch]], sem, add=True)

    def _scatter_wait(ch, g, sem):
        pltpu.make_async_copy(g, accum.at[rows_v.at[ch]], sem).wait()

    def _scale(ch, g):
        base16 = jnp.full((NL,), ch * CHUNK, jnp.int32)

        unroll = 8

        def _body(jo, _):
            j0 = jo * unroll
            for u in range(unroll):
                v16 = plsc.load_gather(vals_v, [base16 + (j0 + u)])
                for q in range(b // NL):
                    g[j0 + u, pl.ds(q * NL, NL)] = (
                        g[j0 + u, pl.ds(q * NL, NL)] * v16)
            return 0
        lax.fori_loop(0, CHUNK // unroll, _body, 0)

    for u in range(NBUF):
        _gather(u, g_bufs[u], sem_g[u])
    last = n_chunks - 1

    def _chunk_body(i, _):
        base = NBUF * i
        for u in range(NBUF):
            ch = base + u
            _gather_wait(ch, g_bufs[u], sem_g[u])
            _scale(ch, g_bufs[u])
            _scatter(ch, g_bufs[u], sem_s[u])
        for u in range(NBUF):
            ch = base + u
            _scatter_wait(ch, g_bufs[u], sem_s[u])
            _gather(jnp.minimum(ch + NBUF, last), g_bufs[u], sem_g[u])
        return 0

    lax.fori_loop(0, n_chunks // NBUF, _chunk_body, 0)
    for u in range(NBUF):
        _gather_wait(last, g_bufs[u], sem_g[u])
    plsc.subcore_barrier()

    pltpu.sync_copy(accum.at[pl.ds(s * rows_per_tile, rows_per_tile)],
                    out_hbm.at[c].at[pl.ds(s * rows_per_tile, rows_per_tile)])


def _merge_body(p_ref, o_ref):
    o_ref[...] = (p_ref[0] + p_ref[1]).T


@jax.jit
def kernel(x, values, indices_float):
    b, cin = x.shape
    nnz = values.shape[0]
    cout = cin

    idx = jnp.round(indices_float).astype(jnp.int32)
    rows, cols = idx[0], idx[1]
    xt = x.T

    c0, c1 = CORE_CHUNKS
    n_chunks = max(c0, c1)
    split = NS * c0 * CHUNK
    total = NS * (c0 + c1) * CHUNK
    assert total >= nnz

    def _shard(a):
        a = jnp.pad(a, (0, total - nnz))
        p0 = jnp.pad(a[:split].reshape(NS, c0, CHUNK),
                     ((0, 0), (0, n_chunks - c0), (0, 0)))
        p1 = jnp.pad(a[split:].reshape(NS, c1, CHUNK),
                     ((0, 0), (0, n_chunks - c1), (0, 0)))
        return jnp.concatenate([p0, p1], axis=0)

    rows_p = _shard(rows)
    cols_p = _shard(cols)
    vals_p = _shard(values).reshape(NW, n_chunks * CHUNK)

    spmm = functools.partial(
        pl.kernel,
        out_type=jax.ShapeDtypeStruct((NC, cout, b), jnp.float32),
        mesh=plsc.VectorSubcoreMesh(core_axis_name="c", subcore_axis_name="s"),
        scratch_types=[
            pltpu.VMEM((n_chunks, CHUNK), jnp.int32),
            pltpu.VMEM((n_chunks, CHUNK), jnp.int32),
            pltpu.VMEM((n_chunks * CHUNK,), jnp.float32),
            [pltpu.VMEM((CHUNK, b), jnp.float32)] * NBUF,
            pltpu.VMEM_SHARED((cout, b), jnp.float32),
            [pltpu.SemaphoreType.DMA] * NBUF,
            [pltpu.SemaphoreType.DMA] * NBUF,
        ],
        compiler_params=pltpu.CompilerParams(needs_layout_passes=False,
                                             use_tc_tiling_on_sc=False),
    )(functools.partial(_spmm_sc_kernel, cout))

    partials = spmm(xt, cols_p, rows_p, vals_p)

    n_blk = 8
    return pl.pallas_call(
        _merge_body,
        out_shape=jax.ShapeDtypeStruct((b, cout), jnp.float32),
        grid=(n_blk,),
        in_specs=[pl.BlockSpec((NC, cout // n_blk, b), lambda i: (0, i, 0))],
        out_specs=pl.BlockSpec((b, cout // n_blk), lambda i: (0, i)),
    )(partials)


# speedup vs baseline: 1.0858x; 1.0858x over previous
import functools

import jax
import jax.numpy as jnp
from jax import lax
from jax.experimental import pallas as pl
from jax.experimental.pallas import tpu as pltpu
from jax.experimental.pallas import tpu_sc as plsc

NC = 2
NS = 16
NL = 16
NW = NC * NS

CHUNK = 128


NBUF = 2

CORE_CHUNKS = (76, 56)


def _spmm_sc_kernel(cout, xt_hbm, cols_hbm, rows_hbm, vals_hbm,
                    out_hbm, cols_v, rows_v, vals_v, g_bufs, accum,
                    sem_g, sem_s):
    c = lax.axis_index("c")
    s = lax.axis_index("s")
    wid = c * NS + s
    n_chunks = jnp.where(c == 0, CORE_CHUNKS[0], CORE_CHUNKS[1])
    g_a = g_bufs[0]
    b = g_a.shape[1]

    zero16 = jnp.zeros((NL,), jnp.float32)

    def _zrow(i, _):
        for q in range(b // NL):
            g_a[i, pl.ds(q * NL, NL)] = zero16
        return 0
    lax.fori_loop(0, CHUNK, _zrow, 0)

    rows_per_tile = cout // NS
    for k in range(rows_per_tile // CHUNK):
        pltpu.sync_copy(g_a, accum.at[pl.ds(s * rows_per_tile + k * CHUNK, CHUNK)])
    plsc.subcore_barrier()

    pltpu.sync_copy(cols_hbm.at[wid], cols_v)
    pltpu.sync_copy(rows_hbm.at[wid], rows_v)
    pltpu.sync_copy(vals_hbm.at[wid], vals_v)

    def _gather(ch, g, sem):
        pltpu.async_copy(xt_hbm.at[cols_v.at[ch]], g, sem)

    def _gather_wait(ch, g, sem):
        pltpu.make_async_copy(xt_hbm.at[cols_v.at[ch]], g, sem).wait()

    def _scatter(ch, g, sem):
        pltpu.async_copy(g, accum.at[rows_v.at[ch]], sem, add=True)

    def _scatter_wait(ch, g, sem):
        pltpu.make_async_copy(g, accum.at[rows_v.at[ch]], sem).wait()

    def _scale(ch, g):
        base16 = jnp.full((NL,), ch * CHUNK, jnp.int32)

        unroll = 8

        def _body(jo, _):
            j0 = jo * unroll
            for u in range(unroll):
                v16 = plsc.load_gather(vals_v, [base16 + (j0 + u)])
                for q in range(b // NL):
                    g[j0 + u, pl.ds(q * NL, NL)] = (
                        g[j0 + u, pl.ds(q * NL, NL)] * v16)
            return 0
        lax.fori_loop(0, CHUNK // unroll, _body, 0)

    for u in range(NBUF):
        _gather(u, g_bufs[u], sem_g[u])
    last = n_chunks - 1

    def _chunk_body(i, _):
        base = NBUF * i
        for u in range(NBUF):
            ch = base + u
            _gather_wait(ch, g_bufs[u], sem_g[u])
            _scale(ch, g_bufs[u])
            _scatter(ch, g_bufs[u], sem_s[u])
        for u in range(NBUF):
            ch = base + u
            _scatter_wait(ch, g_bufs[u], sem_s[u])
            _gather(jnp.minimum(ch + NBUF, last), g_bufs[u], sem_g[u])
        return 0

    lax.fori_loop(0, n_chunks // NBUF, _chunk_body, 0)
    for u in range(NBUF):
        _gather_wait(last, g_bufs[u], sem_g[u])
    plsc.subcore_barrier()

    pltpu.sync_copy(accum.at[pl.ds(s * rows_per_tile, rows_per_tile)],
                    out_hbm.at[c].at[pl.ds(s * rows_per_tile, rows_per_tile)])


def _merge_body(p_ref, o_ref):
    o_ref[...] = (p_ref[0] + p_ref[1]).T


@jax.jit
def kernel(x, values, indices_float):
    b, cin = x.shape
    nnz = values.shape[0]
    cout = cin

    idx = jnp.round(indices_float).astype(jnp.int32)
    rows, cols = idx[0], idx[1]
    xt = x.T

    c0, c1 = CORE_CHUNKS
    n_chunks = max(c0, c1)
    split = NS * c0 * CHUNK
    total = NS * (c0 + c1) * CHUNK
    assert total >= nnz

    def _shard(a):
        a = jnp.pad(a, (0, total - nnz))
        p0 = jnp.pad(a[:split].reshape(NS, c0, CHUNK),
                     ((0, 0), (0, n_chunks - c0), (0, 0)))
        p1 = jnp.pad(a[split:].reshape(NS, c1, CHUNK),
                     ((0, 0), (0, n_chunks - c1), (0, 0)))
        return jnp.concatenate([p0, p1], axis=0)

    rows_p = _shard(rows)
    cols_p = _shard(cols)
    vals_p = _shard(values).reshape(NW, n_chunks * CHUNK)

    spmm = functools.partial(
        pl.kernel,
        out_type=jax.ShapeDtypeStruct((NC, cout, b), jnp.float32),
        mesh=plsc.VectorSubcoreMesh(core_axis_name="c", subcore_axis_name="s"),
        scratch_types=[
            pltpu.VMEM((n_chunks, CHUNK), jnp.int32),
            pltpu.VMEM((n_chunks, CHUNK), jnp.int32),
            pltpu.VMEM((n_chunks * CHUNK,), jnp.float32),
            [pltpu.VMEM((CHUNK, b), jnp.float32)] * NBUF,
            pltpu.VMEM_SHARED((cout, b), jnp.float32),
            [pltpu.SemaphoreType.DMA] * NBUF,
            [pltpu.SemaphoreType.DMA] * NBUF,
        ],
        compiler_params=pltpu.CompilerParams(needs_layout_passes=False,
                                             use_tc_tiling_on_sc=False),
    )(functools.partial(_spmm_sc_kernel, cout))

    partials = spmm(xt, cols_p, rows_p, vals_p)

    n_blk = 8
    return pl.pallas_call(
        _merge_body,
        out_shape=jax.ShapeDtypeStruct((b, cout), jnp.float32),
        grid=(n_blk,),
        in_specs=[pl.BlockSpec((NC, cout // n_blk, b), lambda i: (0, i, 0))],
        out_specs=pl.BlockSpec((b, cout // n_blk), lambda i: (0, i)),
    )(partials)
